# baseline (speedup 1.0000x reference)
import functools

import jax
import jax.numpy as jnp
from jax import lax
from jax.experimental import pallas as pl
from jax.experimental.pallas import tpu as pltpu
from jax.experimental.pallas import tpu_sc as plsc

N = 10000
D = 128
H = 1024
C = 64
B = 4096
E = 320000
EPS = 1e-10

NPAD = 10240
HALF = NPAD // 2
RPT = HALF // 16
NCORES = 2
NSUB = 16
NW = NCORES * NSUB
CHUNK = 128
CHT = 2560
E_PAD = CHT * CHUNK
CH_HIST = CHT // NW
CH_AGG = CHT // NSUB
UV_CH = (2 * B) // (NW * CHUNK)
TRASH = HALF
ACC_ROWS = HALF + 8
CAP = CH_HIST * CHUNK
CAP16 = CAP + 16

F32 = jnp.float32
I32 = jnp.int32

_MESH = plsc.VectorSubcoreMesh(core_axis_name="c", subcore_axis_name="s")


@functools.partial(
    pl.kernel,
    out_type=(
        jax.ShapeDtypeStruct((NW, 1, NPAD), F32),
        jax.ShapeDtypeStruct((2 * B, 128), F32),
        jax.ShapeDtypeStruct((2, NW, CH_HIST, CHUNK), I32),
        jax.ShapeDtypeStruct((2, NW, CH_HIST, CHUNK), I32),
        jax.ShapeDtypeStruct((2, NW, 1, 16), I32),
    ),
    mesh=_MESH,
    compiler_params=pltpu.CompilerParams(needs_layout_passes=False),
    scratch_types=[
        pltpu.VMEM((CH_HIST, CHUNK), I32),
        pltpu.VMEM((CH_HIST, CHUNK), I32),
        pltpu.VMEM((NPAD,), F32),
        pltpu.VMEM((CAP16,), I32),
        pltpu.VMEM((CAP16,), I32),
        pltpu.VMEM((CAP16,), I32),
        pltpu.VMEM((CAP16,), I32),
        pltpu.VMEM((CH_HIST, CHUNK), I32),
        pltpu.VMEM((16,), I32),
        pltpu.VMEM((UV_CH, CHUNK), I32),
        pltpu.VMEM((CHUNK, 128), F32),
        pltpu.SemaphoreType.DMA,
    ],
)
def _sc_hist_gather(dst_hbm, src_hbm, uv_hbm, tab_hbm, zeros_hbm,
                    sfill_hbm, dfill_hbm,
                    hist_out, uvrows_out, srcp_out, dstp_out, cnt_out,
                    didx_v, sidx_v, hist_v, la_s, la_d, lb_s, lb_d,
                    stage_v, cnt_v, uvidx_v, bufa, sem0):
    cid = lax.axis_index("c")
    sid = lax.axis_index("s")
    wid = cid * NSUB + sid
    KPC = CHUNK // 16

    pltpu.sync_copy(zeros_hbm, hist_v)
    pltpu.sync_copy(dst_hbm.at[pl.ds(wid * CH_HIST, CH_HIST)], didx_v)
    pltpu.sync_copy(src_hbm.at[pl.ds(wid * CH_HIST, CH_HIST)], sidx_v)
    pltpu.sync_copy(sfill_hbm, la_s)
    pltpu.sync_copy(dfill_hbm, la_d)
    pltpu.sync_copy(sfill_hbm, lb_s)
    pltpu.sync_copy(dfill_hbm, lb_d)

    pltpu.sync_copy(uv_hbm.at[wid], uvidx_v)
    cp0 = pltpu.async_copy(tab_hbm.at[uvidx_v.at[0]], bufa, sem0)

    ones = jnp.ones((16,), F32)

    def hist_body(i, carry):
        pA, pB = carry
        j = i // KPC
        k = i % KPC
        dv = didx_v[j, pl.ds(k * 16, 16)]
        sv = sidx_v[j, pl.ds(k * 16, 16)]
        plsc.addupdate_scatter(hist_v, [dv], ones)
        mA = dv < HALF
        plsc.store_compressed(la_s.at[pl.ds(pA, 16)], sv, mask=mA)
        plsc.store_compressed(la_d.at[pl.ds(pA, 16)], dv, mask=mA)
        nA = plsc.all_reduce_population_count(mA)[0]
        mB = dv >= HALF
        plsc.store_compressed(lb_s.at[pl.ds(pB, 16)], sv, mask=mB)
        plsc.store_compressed(lb_d.at[pl.ds(pB, 16)], dv - HALF, mask=mB)
        return (pA + nA, pB + (16 - nA))

    pA, pB = lax.fori_loop(0, CH_HIST * KPC, hist_body, (0, 0))
    pltpu.sync_copy(hist_v, hist_out.at[wid, 0])

    npA = jnp.maximum((pA + 2 * CHUNK - 1) // (2 * CHUNK), 1)
    npB = jnp.maximum((pB + 2 * CHUNK - 1) // (2 * CHUNK), 1)
    cnt_v[...] = jnp.full((16,), npA, I32)
    pltpu.sync_copy(cnt_v, cnt_out.at[0, wid, 0])
    cnt_v[...] = jnp.full((16,), npB, I32)
    pltpu.sync_copy(cnt_v, cnt_out.at[1, wid, 0])

    for l1d, out_ref, k in ((la_s, srcp_out, 0), (la_d, dstp_out, 0),
                            (lb_s, srcp_out, 1), (lb_d, dstp_out, 1)):
        def stage_body(i, carry, l1d=l1d):
            stage_v[i // KPC, pl.ds((i % KPC) * 16, 16)] = l1d[pl.ds(i * 16, 16)]
            return carry
        lax.fori_loop(0, CH_HIST * KPC, stage_body, 0)
        pltpu.sync_copy(stage_v, out_ref.at[k, wid])

    cp0.wait()
    pltpu.sync_copy(bufa, uvrows_out.at[pl.ds(wid * UV_CH * CHUNK, CHUNK)])
    for t in range(1, UV_CH):
        pltpu.async_copy(tab_hbm.at[uvidx_v.at[t]], bufa, sem0).wait()
        pltpu.sync_copy(
            bufa, uvrows_out.at[pl.ds((wid * UV_CH + t) * CHUNK, CHUNK)])


@functools.partial(
    pl.kernel,
    out_type=jax.ShapeDtypeStruct((NCORES, HALF, 128), F32),
    mesh=_MESH,
    scratch_types=[
        pltpu.VMEM((CH_HIST, CHUNK), I32),
        pltpu.VMEM((CH_HIST, CHUNK), I32),
        pltpu.VMEM((16,), I32),
        pltpu.VMEM((CHUNK, 128), F32),
        pltpu.VMEM((CHUNK, 128), F32),
        pltpu.VMEM_SHARED((ACC_ROWS, 128), F32),
        pltpu.SemaphoreType.DMA,
        pltpu.SemaphoreType.DMA,
        pltpu.SemaphoreType.DMA,
        pltpu.SemaphoreType.DMA,
    ],
)
def _sc_agg(srcp_hbm, dstp_hbm, cnt_hbm, tab_hbm,
            out_hbm,
            sidx_v, didx_v, cnt_v, bufa, bufb, acc, gsa, gsb, ssa, ssb):
    cid = lax.axis_index("c")
    sid = lax.axis_index("s")

    pltpu.sync_copy(tab_hbm.at[pl.ds(cid * HALF + sid * RPT, RPT)],
                    acc.at[pl.ds(sid * RPT, RPT)])
    plsc.subcore_barrier()

    def gather(j, buf, sem):
        pltpu.async_copy(tab_hbm.at[sidx_v.at[j]], buf, sem)

    def gather_wait(j, buf, sem):
        pltpu.make_async_copy(tab_hbm.at[sidx_v.at[j]], buf, sem).wait()

    def scatter(j, buf, sem):
        pltpu.async_copy(buf, acc.at[didx_v.at[j]], sem, add=True)

    def scatter_wait(j, buf, sem):
        pltpu.make_async_copy(buf, acc.at[didx_v.at[j]], sem).wait()

    for r in range(NCORES):
        p = r * NSUB + sid
        pltpu.sync_copy(srcp_hbm.at[cid, p], sidx_v)
        pltpu.sync_copy(dstp_hbm.at[cid, p], didx_v)
        pltpu.sync_copy(cnt_hbm.at[cid, p, 0], cnt_v)
        npairs = cnt_v[...][0]

        gather(0, bufa, gsa)
        gather(1, bufb, gsb)

        def body(t, carry):
            j0 = t * 2
            gather_wait(j0, bufa, gsa)
            scatter(j0, bufa, ssa)
            gather_wait(j0 + 1, bufb, gsb)
            scatter(j0 + 1, bufb, ssb)
            scatter_wait(j0, bufa, ssa)
            gather(j0 + 2, bufa, gsa)
            scatter_wait(j0 + 1, bufb, ssb)
            gather(j0 + 3, bufb, gsb)
            return carry

        lax.fori_loop(0, npairs - 1, body, 0)

        jl = (npairs - 1) * 2
        gather_wait(jl, bufa, gsa)
        scatter(jl, bufa, ssa)
        gather_wait(jl + 1, bufb, gsb)
        scatter(jl + 1, bufb, ssb)
        scatter_wait(jl, bufa, ssa)
        scatter_wait(jl + 1, bufb, ssb)

    plsc.subcore_barrier()
    pltpu.sync_copy(acc.at[pl.ds(sid * RPT, RPT)],
                    out_hbm.at[cid, pl.ds(sid * RPT, RPT)])


@functools.partial(
    pl.kernel,
    out_type=jax.ShapeDtypeStruct((2 * B, 128), F32),
    mesh=_MESH,
    scratch_types=[
        pltpu.VMEM((UV_CH, CHUNK), I32),
        pltpu.VMEM((CHUNK, 128), F32),
        pltpu.SemaphoreType.DMA,
    ],
)
def _sc_gather_uv(uv_hbm, tab_hbm, out_hbm, uvidx_v, bufa, sem0):
    cid = lax.axis_index("c")
    sid = lax.axis_index("s")
    wid = cid * NSUB + sid
    pltpu.sync_copy(uv_hbm.at[wid], uvidx_v)
    for t in range(UV_CH):
        pltpu.async_copy(tab_hbm.at[uvidx_v.at[t]], bufa, sem0).wait()
        pltpu.sync_copy(
            bufa, out_hbm.at[pl.ds((wid * UV_CH + t) * CHUNK, CHUNK)])


def _tc1_body(h_ref, x_ref, dinv_ref, xs_ref):
    deg = jnp.sum(h_ref[...], axis=1, keepdims=True) + 1.0
    dinv = lax.rsqrt(deg)
    dinv_ref[...] = dinv
    xs_ref[...] = x_ref[...] * dinv


def _tc1(hists, xpad):
    blk = 512
    return pl.pallas_call(
        _tc1_body,
        grid=(NPAD // blk,),
        in_specs=[
            pl.BlockSpec((blk, NW), lambda i: (i, 0)),
            pl.BlockSpec((blk, 128), lambda i: (i, 0)),
        ],
        out_specs=[
            pl.BlockSpec((blk, 1), lambda i: (i, 0)),
            pl.BlockSpec((blk, 128), lambda i: (i, 0)),
        ],
        out_shape=[
            jax.ShapeDtypeStruct((NPAD, 1), F32),
            jax.ShapeDtypeStruct((NPAD, 128), F32),
        ],
    )(hists, xpad)


def _tc2_body(a_ref, dinv_ref,
              Wt1_ref, bt1_ref, Wp1_ref, bp1_ref, Wt2_ref, Wp2_ref, zs_ref):
    agg1 = a_ref[...] * dinv_ref[...]
    t = jnp.maximum(
        jnp.dot(agg1, Wt1_ref[...], preferred_element_type=F32) + bt1_ref[...], 0.0)
    p = jnp.maximum(
        jnp.dot(agg1, Wp1_ref[...], preferred_element_type=F32) + bp1_ref[...], 0.0)
    zt = jnp.dot(t, Wt2_ref[...], preferred_element_type=F32)
    zp = jnp.dot(p, Wp2_ref[...], preferred_element_type=F32)
    zs_ref[...] = jnp.concatenate([zt, zp], axis=1) * dinv_ref[...]


def _tc2(agg, dinv, Wt1, bt1, Wp1, bp1, Wt2, Wp2):
    blk = 512
    grid = (NPAD // blk,)
    row = lambda i: (i, 0)
    full2 = lambda i: (0, 0)
    full1 = lambda i: (0,)
    return pl.pallas_call(
        _tc2_body,
        grid=grid,
        in_specs=[
            pl.BlockSpec((blk, 128), row),
            pl.BlockSpec((blk, 1), row),
            pl.BlockSpec((D, H), full2),
            pl.BlockSpec((H,), full1),
            pl.BlockSpec((D, H), full2),
            pl.BlockSpec((H,), full1),
            pl.BlockSpec((H, C), full2),
            pl.BlockSpec((H, C), full2),
        ],
        out_specs=pl.BlockSpec((blk, 128), row),
        out_shape=jax.ShapeDtypeStruct((NPAD, 128), F32),
    )(agg, dinv, Wt1, bt1, Wp1, bp1, Wt2, Wp2)


def _tc3_body(a_ref, dinv_ref, bt2_ref, bp2_ref, T_ref):
    a2 = a_ref[...] * dinv_ref[...]
    lt = a2[:, :C] + bt2_ref[...]
    lt = lt - jnp.max(lt, axis=1, keepdims=True)
    et = jnp.exp(lt)
    th = et / jnp.sum(et, axis=1, keepdims=True)
    ph = jax.nn.sigmoid(a2[:, C:] + bp2_ref[...])
    T_ref[...] = jnp.concatenate([th, ph], axis=1)


def _tc3(agg, dinv, bt2, bp2):
    blk = 512
    grid = (NPAD // blk,)
    return pl.pallas_call(
        _tc3_body,
        grid=grid,
        in_specs=[
            pl.BlockSpec((blk, 128), lambda i: (i, 0)),
            pl.BlockSpec((blk, 1), lambda i: (i, 0)),
            pl.BlockSpec((C,), lambda i: (0,)),
            pl.BlockSpec((C,), lambda i: (0,)),
        ],
        out_specs=pl.BlockSpec((blk, 128), lambda i: (i, 0)),
        out_shape=jax.ShapeDtypeStruct((NPAD, 128), F32),
    )(agg, dinv, bt2, bp2)


def _tc4_body(xu_ref, xv_ref, Tu_ref, Tv_ref,
              W1a_ref, W1b_ref, b1_ref, W2_ref, b2_ref, W3_ref, b3_ref,
              etaP_ref, q_ref, p_ref, eta_ref):
    h1 = jnp.dot(xu_ref[...], W1a_ref[...], preferred_element_type=F32)
    h1 = h1 + jnp.dot(xv_ref[...], W1b_ref[...], preferred_element_type=F32)
    h1 = jnp.maximum(h1 + b1_ref[...], 0.0)
    h2 = jnp.maximum(
        jnp.dot(h1, W2_ref[...], preferred_element_type=F32) + b2_ref[...], 0.0)
    l3 = jnp.dot(h2, W3_ref[...], preferred_element_type=F32) + b3_ref[...]
    l3 = l3 - jnp.max(l3, axis=1, keepdims=True)
    e3 = jnp.exp(l3)
    q_ref[...] = e3 / jnp.sum(e3, axis=1, keepdims=True) + EPS

    eta = jnp.tanh(etaP_ref[...])
    ae = jnp.abs(eta)
    p_ref[...] = (ae * Tu_ref[:, :C] * Tv_ref[:, :C]
                  + (1.0 - ae) * Tu_ref[:, C:] * Tv_ref[:, C:] + EPS)
    eta_ref[...] = eta


def _tc4(xu, xv, Tu, Tv, W1a, W1b, b1, W2, b2, W3, b3, etaP):
    blk = 512
    grid = (B // blk,)
    row = lambda i: (i, 0)
    full2 = lambda i: (0, 0)
    full1 = lambda i: (0,)
    return pl.pallas_call(
        _tc4_body,
        grid=grid,
        in_specs=[
            pl.BlockSpec((blk, 128), row),
            pl.BlockSpec((blk, 128), row),
            pl.BlockSpec((blk, 128), row),
            pl.BlockSpec((blk, 128), row),
            pl.BlockSpec((128, 1024), full2),
            pl.BlockSpec((128, 1024), full2),
            pl.BlockSpec((1024,), full1),
            pl.BlockSpec((1024, 256), full2),
            pl.BlockSpec((256,), full1),
            pl.BlockSpec((256, C), full2),
            pl.BlockSpec((C,), full1),
            pl.BlockSpec((C,), full1),
        ],
        out_specs=[
            pl.BlockSpec((blk, C), row),
            pl.BlockSpec((blk, C), row),
            pl.BlockSpec((C,), full1),
        ],
        out_shape=[
            jax.ShapeDtypeStruct((B, C), F32),
            jax.ShapeDtypeStruct((B, C), F32),
            jax.ShapeDtypeStruct((C,), F32),
        ],
    )(xu, xv, Tu, Tv, W1a, W1b, b1, W2, b2, W3, b3, etaP)


def kernel(u, v, edge_index, node_features, link_neighbors, eta_param,
           Wt1, bt1, Wt2, bt2, Wp1, bp1, Wp2, bp2,
           Wq1, bq1, Wq2, bq2, Wq3, bq3):
    src = edge_index[0].astype(I32)
    dst = edge_index[1].astype(I32)
    pad = jnp.full((E_PAD - E,), N, I32)
    src2d = jnp.concatenate([src, pad]).reshape(CHT, CHUNK)
    dst2d = jnp.concatenate([dst, pad]).reshape(CHT, CHUNK)
    uv2d = jnp.concatenate([u.astype(I32), v.astype(I32)]).reshape(
        NW, UV_CH, CHUNK)
    xpad = jnp.pad(node_features, ((0, NPAD - N), (0, 0)))
    zerosN = jnp.zeros((NPAD,), F32)
    sfill = jnp.full((CAP16,), N, I32)
    dfill = jnp.full((CAP16,), TRASH, I32)

    hists, uvrows, srcp, dstp, cnts = _sc_hist_gather(
        dst2d, src2d, uv2d, link_neighbors, zerosN, sfill, dfill)
    histsT = hists.reshape(NW, NPAD).T

    dinv, xs = _tc1(histsT, xpad)

    agg1 = _sc_agg(srcp, dstp, cnts, xs).reshape(NPAD, 128)
    zs = _tc2(agg1, dinv, Wt1, bt1, Wp1, bp1, Wt2, Wp2)

    agg2 = _sc_agg(srcp, dstp, cnts, zs).reshape(NPAD, 128)
    T = _tc3(agg2, dinv, bt2, bp2)

    TuTv = _sc_gather_uv(uv2d, T)

    W1p = jnp.pad(Wq1, ((0, 0), (0, 1024 - Wq1.shape[1])))
    b1p = jnp.pad(bq1, (0, 1024 - bq1.shape[0]))
    W2p = jnp.pad(Wq2, ((0, 1024 - Wq2.shape[0]), (0, 256 - Wq2.shape[1])))
    b2p = jnp.pad(bq2, (0, 256 - bq2.shape[0]))
    W3p = jnp.pad(Wq3, ((0, 256 - Wq3.shape[0]), (0, 0)))

    q_probs, p_probs, eta = _tc4(
        uvrows[:B], uvrows[B:], TuTv[:B], TuTv[B:],
        W1p[:128], W1p[128:], b1p, W2p, b2p, W3p, bq3, eta_param)

    return (q_probs, p_probs, eta)


# speedup vs baseline: 1.8141x; 1.8141x over previous
import functools

import jax
import jax.numpy as jnp
from jax import lax
from jax.experimental import pallas as pl
from jax.experimental.pallas import tpu as pltpu
from jax.experimental.pallas import tpu_sc as plsc

N = 10000
D = 128
H = 1024
C = 64
B = 4096
E = 320000
EPS = 1e-10

NPAD = 10240
HALF = NPAD // 2
RPT = HALF // 16
NCORES = 2
NSUB = 16
NW = NCORES * NSUB
CHUNK = 128
CHT = 2560
E_PAD = CHT * CHUNK
CH_HIST = CHT // NW
CH_AGG = CHT // NSUB
UV_CH = (2 * B) // (NW * CHUNK)
TRASH = HALF
ACC_ROWS = HALF + 8
CAP = CH_HIST * CHUNK
CAP16 = CAP + 16

F32 = jnp.float32
I32 = jnp.int32

_MESH = plsc.VectorSubcoreMesh(core_axis_name="c", subcore_axis_name="s")


@functools.partial(
    pl.kernel,
    out_type=(
        jax.ShapeDtypeStruct((NW, 1, NPAD), F32),
        jax.ShapeDtypeStruct((2 * B, 128), F32),
        jax.ShapeDtypeStruct((2, NW, CH_HIST, CHUNK), I32),
        jax.ShapeDtypeStruct((2, NW, CH_HIST, CHUNK), I32),
        jax.ShapeDtypeStruct((2, NW, 1, 16), I32),
    ),
    mesh=_MESH,
    compiler_params=pltpu.CompilerParams(needs_layout_passes=False),
    scratch_types=[
        pltpu.VMEM((CH_HIST, CHUNK), I32),
        pltpu.VMEM((CH_HIST, CHUNK), I32),
        pltpu.VMEM((NPAD + 16,), F32),
        pltpu.VMEM((CAP16,), I32),
        pltpu.VMEM((CAP16,), I32),
        pltpu.VMEM((CAP16,), I32),
        pltpu.VMEM((CAP16,), I32),
        pltpu.VMEM((CH_HIST, CHUNK), I32),
        pltpu.VMEM((16,), I32),
        pltpu.VMEM((UV_CH, CHUNK), I32),
        pltpu.VMEM((CHUNK, 128), F32),
        pltpu.SemaphoreType.DMA,
    ],
)
def _sc_hist_gather(dst_hbm, src_hbm, uv_hbm, tab_hbm, zeros_hbm,
                    sfill_hbm, dfill_hbm,
                    hist_out, uvrows_out, srcp_out, dstp_out, cnt_out,
                    didx_v, sidx_v, hist_v, la_s, la_d, lb_s, lb_d,
                    stage_v, cnt_v, uvidx_v, bufa, sem0):
    cid = lax.axis_index("c")
    sid = lax.axis_index("s")
    wid = cid * NSUB + sid
    KPC = CHUNK // 16

    pltpu.sync_copy(zeros_hbm, hist_v)
    pltpu.sync_copy(dst_hbm.at[pl.ds(wid * CH_HIST, CH_HIST)], didx_v)
    pltpu.sync_copy(src_hbm.at[pl.ds(wid * CH_HIST, CH_HIST)], sidx_v)
    pltpu.sync_copy(sfill_hbm, la_s)
    pltpu.sync_copy(dfill_hbm, la_d)
    pltpu.sync_copy(sfill_hbm, lb_s)
    pltpu.sync_copy(dfill_hbm, lb_d)

    pltpu.sync_copy(uv_hbm.at[wid], uvidx_v)
    cp0 = pltpu.async_copy(tab_hbm.at[uvidx_v.at[0]], bufa, sem0)

    ones = jnp.ones((16,), F32)

    def hist_body(i, carry):
        pA, pB = carry
        j = i // KPC
        k = i % KPC
        dv = didx_v[j, pl.ds(k * 16, 16)]
        sv = sidx_v[j, pl.ds(k * 16, 16)]
        plsc.addupdate_scatter(hist_v, [dv], ones)
        mA = dv < HALF
        plsc.store_compressed(la_s.at[pl.ds(pA, 16)], sv, mask=mA)
        plsc.store_compressed(la_d.at[pl.ds(pA, 16)], dv, mask=mA)
        nA = plsc.all_reduce_population_count(mA)[0]
        mB = jnp.logical_and(dv >= HALF, dv < NPAD)
        plsc.store_compressed(lb_s.at[pl.ds(pB, 16)], sv, mask=mB)
        plsc.store_compressed(lb_d.at[pl.ds(pB, 16)], dv - HALF, mask=mB)
        nB = plsc.all_reduce_population_count(mB)[0]
        return (pA + nA, pB + nB)

    pA, pB = lax.fori_loop(0, CH_HIST * KPC, hist_body, (0, 0))
    pltpu.sync_copy(hist_v.at[pl.ds(0, NPAD)], hist_out.at[wid, 0])

    npA = jnp.maximum((pA + 2 * CHUNK - 1) // (2 * CHUNK), 1)
    npB = jnp.maximum((pB + 2 * CHUNK - 1) // (2 * CHUNK), 1)
    cnt_v[...] = jnp.full((16,), npA, I32)
    pltpu.sync_copy(cnt_v, cnt_out.at[0, wid, 0])
    cnt_v[...] = jnp.full((16,), npB, I32)
    pltpu.sync_copy(cnt_v, cnt_out.at[1, wid, 0])

    for l1d, out_ref, k in ((la_s, srcp_out, 0), (la_d, dstp_out, 0),
                            (lb_s, srcp_out, 1), (lb_d, dstp_out, 1)):
        def stage_body(i, carry, l1d=l1d):
            stage_v[i // KPC, pl.ds((i % KPC) * 16, 16)] = l1d[pl.ds(i * 16, 16)]
            return carry
        lax.fori_loop(0, CH_HIST * KPC, stage_body, 0)
        pltpu.sync_copy(stage_v, out_ref.at[k, wid])

    cp0.wait()
    pltpu.sync_copy(bufa, uvrows_out.at[pl.ds(wid * UV_CH * CHUNK, CHUNK)])
    for t in range(1, UV_CH):
        pltpu.async_copy(tab_hbm.at[uvidx_v.at[t]], bufa, sem0).wait()
        pltpu.sync_copy(
            bufa, uvrows_out.at[pl.ds((wid * UV_CH + t) * CHUNK, CHUNK)])


@functools.partial(
    pl.kernel,
    out_type=jax.ShapeDtypeStruct((NCORES, HALF, 128), F32),
    mesh=_MESH,
    scratch_types=[
        pltpu.VMEM((CH_HIST, CHUNK), I32),
        pltpu.VMEM((CH_HIST, CHUNK), I32),
        pltpu.VMEM((16,), I32),
        pltpu.VMEM((CHUNK, 128), F32),
        pltpu.VMEM((CHUNK, 128), F32),
        pltpu.VMEM_SHARED((ACC_ROWS, 128), F32),
        pltpu.SemaphoreType.DMA,
        pltpu.SemaphoreType.DMA,
        pltpu.SemaphoreType.DMA,
        pltpu.SemaphoreType.DMA,
    ],
)
def _sc_agg(srcp_hbm, dstp_hbm, cnt_hbm, tab_hbm,
            out_hbm,
            sidx_v, didx_v, cnt_v, bufa, bufb, acc, gsa, gsb, ssa, ssb):
    cid = lax.axis_index("c")
    sid = lax.axis_index("s")

    pltpu.sync_copy(tab_hbm.at[pl.ds(cid * HALF + sid * RPT, RPT)],
                    acc.at[pl.ds(sid * RPT, RPT)])
    plsc.subcore_barrier()

    def gather(j, buf, sem):
        pltpu.async_copy(tab_hbm.at[sidx_v.at[j]], buf, sem)

    def gather_wait(j, buf, sem):
        pltpu.make_async_copy(tab_hbm.at[sidx_v.at[j]], buf, sem).wait()

    def scatter(j, buf, sem):
        pltpu.async_copy(buf, acc.at[didx_v.at[j]], sem, add=True)

    def scatter_wait(j, buf, sem):
        pltpu.make_async_copy(buf, acc.at[didx_v.at[j]], sem).wait()

    for r in range(NCORES):
        p = r * NSUB + sid
        pltpu.sync_copy(srcp_hbm.at[cid, p], sidx_v)
        pltpu.sync_copy(dstp_hbm.at[cid, p], didx_v)
        pltpu.sync_copy(cnt_hbm.at[cid, p, 0], cnt_v)
        npairs = cnt_v[...][0]

        gather(0, bufa, gsa)
        gather(1, bufb, gsb)

        def body(t, carry):
            j0 = t * 2
            gather_wait(j0, bufa, gsa)
            scatter(j0, bufa, ssa)
            gather_wait(j0 + 1, bufb, gsb)
            scatter(j0 + 1, bufb, ssb)
            scatter_wait(j0, bufa, ssa)
            gather(j0 + 2, bufa, gsa)
            scatter_wait(j0 + 1, bufb, ssb)
            gather(j0 + 3, bufb, gsb)
            return carry

        lax.fori_loop(0, npairs - 1, body, 0)

        jl = (npairs - 1) * 2
        gather_wait(jl, bufa, gsa)
        scatter(jl, bufa, ssa)
        gather_wait(jl + 1, bufb, gsb)
        scatter(jl + 1, bufb, ssb)
        scatter_wait(jl, bufa, ssa)
        scatter_wait(jl + 1, bufb, ssb)

    plsc.subcore_barrier()
    pltpu.sync_copy(acc.at[pl.ds(sid * RPT, RPT)],
                    out_hbm.at[cid, pl.ds(sid * RPT, RPT)])


@functools.partial(
    pl.kernel,
    out_type=jax.ShapeDtypeStruct((2 * B, 128), F32),
    mesh=_MESH,
    scratch_types=[
        pltpu.VMEM((UV_CH, CHUNK), I32),
        pltpu.VMEM((CHUNK, 128), F32),
        pltpu.SemaphoreType.DMA,
    ],
)
def _sc_gather_uv(uv_hbm, tab_hbm, out_hbm, uvidx_v, bufa, sem0):
    cid = lax.axis_index("c")
    sid = lax.axis_index("s")
    wid = cid * NSUB + sid
    pltpu.sync_copy(uv_hbm.at[wid], uvidx_v)
    for t in range(UV_CH):
        pltpu.async_copy(tab_hbm.at[uvidx_v.at[t]], bufa, sem0).wait()
        pltpu.sync_copy(
            bufa, out_hbm.at[pl.ds((wid * UV_CH + t) * CHUNK, CHUNK)])


def _tc1_body(h_ref, x_ref, dinv_ref, xs_ref):
    deg = jnp.sum(h_ref[...], axis=1, keepdims=True) + 1.0
    dinv = lax.rsqrt(deg)
    dinv_ref[...] = dinv
    xs_ref[...] = x_ref[...] * dinv


def _tc1(hists, xpad):
    blk = 512
    return pl.pallas_call(
        _tc1_body,
        grid=(NPAD // blk,),
        in_specs=[
            pl.BlockSpec((blk, NW), lambda i: (i, 0)),
            pl.BlockSpec((blk, 128), lambda i: (i, 0)),
        ],
        out_specs=[
            pl.BlockSpec((blk, 1), lambda i: (i, 0)),
            pl.BlockSpec((blk, 128), lambda i: (i, 0)),
        ],
        out_shape=[
            jax.ShapeDtypeStruct((NPAD, 1), F32),
            jax.ShapeDtypeStruct((NPAD, 128), F32),
        ],
    )(hists, xpad)


def _tc2_body(a_ref, dinv_ref,
              Wt1_ref, bt1_ref, Wp1_ref, bp1_ref, Wt2_ref, Wp2_ref, zs_ref):
    agg1 = a_ref[...] * dinv_ref[...]
    t = jnp.maximum(
        jnp.dot(agg1, Wt1_ref[...], preferred_element_type=F32) + bt1_ref[...], 0.0)
    p = jnp.maximum(
        jnp.dot(agg1, Wp1_ref[...], preferred_element_type=F32) + bp1_ref[...], 0.0)
    zt = jnp.dot(t, Wt2_ref[...], preferred_element_type=F32)
    zp = jnp.dot(p, Wp2_ref[...], preferred_element_type=F32)
    zs_ref[...] = jnp.concatenate([zt, zp], axis=1) * dinv_ref[...]


def _tc2(agg, dinv, Wt1, bt1, Wp1, bp1, Wt2, Wp2):
    blk = 512
    grid = (NPAD // blk,)
    row = lambda i: (i, 0)
    full2 = lambda i: (0, 0)
    full1 = lambda i: (0,)
    return pl.pallas_call(
        _tc2_body,
        grid=grid,
        in_specs=[
            pl.BlockSpec((blk, 128), row),
            pl.BlockSpec((blk, 1), row),
            pl.BlockSpec((D, H), full2),
            pl.BlockSpec((H,), full1),
            pl.BlockSpec((D, H), full2),
            pl.BlockSpec((H,), full1),
            pl.BlockSpec((H, C), full2),
            pl.BlockSpec((H, C), full2),
        ],
        out_specs=pl.BlockSpec((blk, 128), row),
        out_shape=jax.ShapeDtypeStruct((NPAD, 128), F32),
    )(agg, dinv, Wt1, bt1, Wp1, bp1, Wt2, Wp2)


def _tc3_body(a_ref, dinv_ref, bt2_ref, bp2_ref, T_ref):
    a2 = a_ref[...] * dinv_ref[...]
    lt = a2[:, :C] + bt2_ref[...]
    lt = lt - jnp.max(lt, axis=1, keepdims=True)
    et = jnp.exp(lt)
    th = et / jnp.sum(et, axis=1, keepdims=True)
    ph = jax.nn.sigmoid(a2[:, C:] + bp2_ref[...])
    T_ref[...] = jnp.concatenate([th, ph], axis=1)


def _tc3(agg, dinv, bt2, bp2):
    blk = 512
    grid = (NPAD // blk,)
    return pl.pallas_call(
        _tc3_body,
        grid=grid,
        in_specs=[
            pl.BlockSpec((blk, 128), lambda i: (i, 0)),
            pl.BlockSpec((blk, 1), lambda i: (i, 0)),
            pl.BlockSpec((C,), lambda i: (0,)),
            pl.BlockSpec((C,), lambda i: (0,)),
        ],
        out_specs=pl.BlockSpec((blk, 128), lambda i: (i, 0)),
        out_shape=jax.ShapeDtypeStruct((NPAD, 128), F32),
    )(agg, dinv, bt2, bp2)


def _tc4_body(xu_ref, xv_ref, Tu_ref, Tv_ref,
              W1a_ref, W1b_ref, b1_ref, W2_ref, b2_ref, W3_ref, b3_ref,
              etaP_ref, q_ref, p_ref, eta_ref):
    h1 = jnp.dot(xu_ref[...], W1a_ref[...], preferred_element_type=F32)
    h1 = h1 + jnp.dot(xv_ref[...], W1b_ref[...], preferred_element_type=F32)
    h1 = jnp.maximum(h1 + b1_ref[...], 0.0)
    h2 = jnp.maximum(
        jnp.dot(h1, W2_ref[...], preferred_element_type=F32) + b2_ref[...], 0.0)
    l3 = jnp.dot(h2, W3_ref[...], preferred_element_type=F32) + b3_ref[...]
    l3 = l3 - jnp.max(l3, axis=1, keepdims=True)
    e3 = jnp.exp(l3)
    q_ref[...] = e3 / jnp.sum(e3, axis=1, keepdims=True) + EPS

    eta = jnp.tanh(etaP_ref[...])
    ae = jnp.abs(eta)
    p_ref[...] = (ae * Tu_ref[:, :C] * Tv_ref[:, :C]
                  + (1.0 - ae) * Tu_ref[:, C:] * Tv_ref[:, C:] + EPS)
    eta_ref[...] = eta


def _tc4(xu, xv, Tu, Tv, W1a, W1b, b1, W2, b2, W3, b3, etaP):
    blk = 512
    grid = (B // blk,)
    row = lambda i: (i, 0)
    full2 = lambda i: (0, 0)
    full1 = lambda i: (0,)
    return pl.pallas_call(
        _tc4_body,
        grid=grid,
        in_specs=[
            pl.BlockSpec((blk, 128), row),
            pl.BlockSpec((blk, 128), row),
            pl.BlockSpec((blk, 128), row),
            pl.BlockSpec((blk, 128), row),
            pl.BlockSpec((128, 1024), full2),
            pl.BlockSpec((128, 1024), full2),
            pl.BlockSpec((1024,), full1),
            pl.BlockSpec((1024, 256), full2),
            pl.BlockSpec((256,), full1),
            pl.BlockSpec((256, C), full2),
            pl.BlockSpec((C,), full1),
            pl.BlockSpec((C,), full1),
        ],
        out_specs=[
            pl.BlockSpec((blk, C), row),
            pl.BlockSpec((blk, C), row),
            pl.BlockSpec((C,), full1),
        ],
        out_shape=[
            jax.ShapeDtypeStruct((B, C), F32),
            jax.ShapeDtypeStruct((B, C), F32),
            jax.ShapeDtypeStruct((C,), F32),
        ],
    )(xu, xv, Tu, Tv, W1a, W1b, b1, W2, b2, W3, b3, etaP)


def kernel(u, v, edge_index, node_features, link_neighbors, eta_param,
           Wt1, bt1, Wt2, bt2, Wp1, bp1, Wp2, bp2,
           Wq1, bq1, Wq2, bq2, Wq3, bq3):
    src = edge_index[0].astype(I32)
    dst = edge_index[1].astype(I32)
    pad = jnp.full((E_PAD - E,), N, I32)
    dpad = jnp.full((E_PAD - E,), NPAD, I32)
    src2d = jnp.concatenate([src, pad]).reshape(CHT, CHUNK)
    dst2d = jnp.concatenate([dst, dpad]).reshape(CHT, CHUNK)
    uv2d = jnp.concatenate([u.astype(I32), v.astype(I32)]).reshape(
        NW, UV_CH, CHUNK)
    xpad = jnp.pad(node_features, ((0, NPAD - N), (0, 0)))
    zerosN = jnp.zeros((NPAD + 16,), F32)
    sfill = jnp.full((CAP16,), N, I32)
    dfill = TRASH + (jnp.arange(CAP16, dtype=I32) % 8)

    hists, uvrows, srcp, dstp, cnts = _sc_hist_gather(
        dst2d, src2d, uv2d, link_neighbors, zerosN, sfill, dfill)
    histsT = hists.reshape(NW, NPAD).T

    dinv, xs = _tc1(histsT, xpad)

    agg1 = _sc_agg(srcp, dstp, cnts, xs).reshape(NPAD, 128)
    zs = _tc2(agg1, dinv, Wt1, bt1, Wp1, bp1, Wt2, Wp2)

    agg2 = _sc_agg(srcp, dstp, cnts, zs).reshape(NPAD, 128)
    T = _tc3(agg2, dinv, bt2, bp2)

    TuTv = _sc_gather_uv(uv2d, T)

    W1p = jnp.pad(Wq1, ((0, 0), (0, 1024 - Wq1.shape[1])))
    b1p = jnp.pad(bq1, (0, 1024 - bq1.shape[0]))
    W2p = jnp.pad(Wq2, ((0, 1024 - Wq2.shape[0]), (0, 256 - Wq2.shape[1])))
    b2p = jnp.pad(bq2, (0, 256 - bq2.shape[0]))
    W3p = jnp.pad(Wq3, ((0, 256 - Wq3.shape[0]), (0, 0)))

    q_probs, p_probs, eta = _tc4(
        uvrows[:B], uvrows[B:], TuTv[:B], TuTv[B:],
        W1p[:128], W1p[128:], b1p, W2p, b2p, W3p, bq3, eta_param)

    return (q_probs, p_probs, eta)
